# baseline (speedup 1.0000x reference)
import functools

import jax
import jax.numpy as jnp
from jax import lax
from jax.experimental import pallas as pl
from jax.experimental.pallas import tpu as pltpu
from jax.experimental.pallas import tpu_sc as plsc

L = 16
NC = 2
NS = 16
NW = NC * NS
B = 16384
D = 64
BPW = B // NW
CH = 32
NCH2 = BPW // CH
TB = 8


def _dot_kernel(user_ids, item_ids, uf, itf, ib):
    mesh = plsc.VectorSubcoreMesh(core_axis_name="c", subcore_axis_name="s")

    @functools.partial(
        pl.kernel,
        out_type=jax.ShapeDtypeStruct((B,), jnp.float32),
        mesh=mesh,
        compiler_params=pltpu.CompilerParams(needs_layout_passes=False),
        scratch_types=[
            pltpu.VMEM((NCH2, CH), jnp.int32),
            pltpu.VMEM((NCH2, CH), jnp.int32),
            pltpu.VMEM((CH, TB, D), jnp.float32),
            pltpu.VMEM((CH, TB, D), jnp.float32),
            pltpu.VMEM((CH, TB, 1), jnp.float32),
            pltpu.VMEM((BPW,), jnp.float32),
            pltpu.SemaphoreType.DMA,
        ],
    )
    def run(uids_hbm, iids_hbm, uf_hbm, if_hbm, ib_hbm, out_hbm,
            uidx, iidx, ublocks, iblocks, bblocks, outv, sem):
        wid = lax.axis_index("s") * NC + lax.axis_index("c")
        base = wid * BPW

        for c in range(NCH2):
            pltpu.sync_copy(uids_hbm.at[pl.ds(base + c * CH, CH)],
                            uidx.at[c])
            pltpu.sync_copy(iids_hbm.at[pl.ds(base + c * CH, CH)],
                            iidx.at[c])

        def chunk_body(c, carry):
            for g in range(CH // L):
                ustart = jnp.bitwise_and(uidx[c, pl.ds(g * L, L)], ~7)
                istart = jnp.bitwise_and(iidx[c, pl.ds(g * L, L)], ~7)
                for j in range(L):
                    r = g * L + j
                    us = pl.multiple_of(ustart[j], TB)
                    s = pl.multiple_of(istart[j], TB)
                    pltpu.make_async_copy(
                        uf_hbm.at[pl.ds(us, TB), :],
                        ublocks.at[r], sem).start()
                    pltpu.make_async_copy(
                        if_hbm.at[pl.ds(s, TB), :],
                        iblocks.at[r], sem).start()
                    pltpu.make_async_copy(
                        ib_hbm.at[pl.ds(s, TB), :],
                        bblocks.at[r], sem).start()
            for r in range(CH):
                pltpu.make_async_copy(
                    uf_hbm.at[pl.ds(0, TB), :], ublocks.at[r], sem).wait()
                pltpu.make_async_copy(
                    if_hbm.at[pl.ds(0, TB), :], iblocks.at[r], sem).wait()
                pltpu.make_async_copy(
                    ib_hbm.at[pl.ds(0, TB), :], bblocks.at[r], sem).wait()

            zeros = jnp.zeros((L,), jnp.int32)
            for g in range(CH // L):
                sl = pl.ds(g * L, L)
                jvec = lax.iota(jnp.int32, L) + g * L
                urow = jnp.bitwise_and(uidx[c, sl], 7)
                irow = jnp.bitwise_and(iidx[c, sl], 7)
                acc = plsc.load_gather(bblocks, [jvec, irow, zeros])
                for d in range(D):
                    col = jnp.full((L,), d, jnp.int32)
                    u = plsc.load_gather(ublocks, [jvec, urow, col])
                    it = plsc.load_gather(iblocks, [jvec, irow, col])
                    acc = acc + u * it
                outv[pl.ds(c * CH + g * L, L)] = acc
            return carry

        lax.fori_loop(0, NCH2, chunk_body, 0)
        pltpu.sync_copy(outv, out_hbm.at[pl.ds(base, BPW)])

    return run(user_ids, item_ids, uf, itf, ib)


def kernel(user_ids, item_ids, user_factors, item_factors, item_bias):
    return _dot_kernel(user_ids, item_ids, user_factors, item_factors,
                       item_bias)


# speedup vs baseline: 1.0085x; 1.0085x over previous
import functools

import jax
import jax.numpy as jnp
from jax import lax
from jax.experimental import pallas as pl
from jax.experimental.pallas import tpu as pltpu
from jax.experimental.pallas import tpu_sc as plsc

L = 16
NC = 2
NS = 16
NW = NC * NS
B = 16384
D = 64
BPW = B // NW
CH = 32
NCH2 = BPW // CH
TB = 8


def _dot_kernel(user_ids, item_ids, uf, itf, ib):
    mesh = plsc.VectorSubcoreMesh(core_axis_name="c", subcore_axis_name="s")

    @functools.partial(
        pl.kernel,
        out_type=jax.ShapeDtypeStruct((B,), jnp.float32),
        mesh=mesh,
        compiler_params=pltpu.CompilerParams(needs_layout_passes=False, use_tc_tiling_on_sc=True),
        scratch_types=[
            pltpu.VMEM((NCH2, CH), jnp.int32),
            pltpu.VMEM((NCH2, CH), jnp.int32),
            pltpu.VMEM((CH, TB, D), jnp.float32),
            pltpu.VMEM((CH, TB, D), jnp.float32),
            pltpu.VMEM((CH, TB, 1), jnp.float32),
            pltpu.VMEM((BPW,), jnp.float32),
            pltpu.SemaphoreType.DMA,
        ],
    )
    def run(uids_hbm, iids_hbm, uf_hbm, if_hbm, ib_hbm, out_hbm,
            uidx, iidx, ublocks, iblocks, bblocks, outv, sem):
        wid = lax.axis_index("s") * NC + lax.axis_index("c")
        base = wid * BPW

        for c in range(NCH2):
            pltpu.sync_copy(uids_hbm.at[pl.ds(base + c * CH, CH)],
                            uidx.at[c])
            pltpu.sync_copy(iids_hbm.at[pl.ds(base + c * CH, CH)],
                            iidx.at[c])

        def chunk_body(c, carry):
            for g in range(CH // L):
                ustart = jnp.bitwise_and(uidx[c, pl.ds(g * L, L)], ~7)
                istart = jnp.bitwise_and(iidx[c, pl.ds(g * L, L)], ~7)
                for j in range(L):
                    r = g * L + j
                    us = pl.multiple_of(ustart[j], TB)
                    s = pl.multiple_of(istart[j], TB)
                    pltpu.make_async_copy(
                        uf_hbm.at[pl.ds(us, TB), :],
                        ublocks.at[r], sem).start()
                    pltpu.make_async_copy(
                        if_hbm.at[pl.ds(s, TB), :],
                        iblocks.at[r], sem).start()
                    pltpu.make_async_copy(
                        ib_hbm.at[pl.ds(s, TB), :],
                        bblocks.at[r], sem).start()
            for r in range(CH):
                pltpu.make_async_copy(
                    uf_hbm.at[pl.ds(0, TB), :], ublocks.at[r], sem).wait()
                pltpu.make_async_copy(
                    if_hbm.at[pl.ds(0, TB), :], iblocks.at[r], sem).wait()
                pltpu.make_async_copy(
                    ib_hbm.at[pl.ds(0, TB), :], bblocks.at[r], sem).wait()

            zeros = jnp.zeros((L,), jnp.int32)
            for g in range(CH // L):
                sl = pl.ds(g * L, L)
                jvec = lax.iota(jnp.int32, L) + g * L
                urow = jnp.bitwise_and(uidx[c, sl], 7)
                irow = jnp.bitwise_and(iidx[c, sl], 7)
                acc = plsc.load_gather(bblocks, [jvec, irow, zeros])
                for d in range(D):
                    col = jnp.full((L,), d, jnp.int32)
                    u = plsc.load_gather(ublocks, [jvec, urow, col])
                    it = plsc.load_gather(iblocks, [jvec, irow, col])
                    acc = acc + u * it
                outv[pl.ds(c * CH + g * L, L)] = acc
            return carry

        lax.fori_loop(0, NCH2, chunk_body, 0)
        pltpu.sync_copy(outv, out_hbm.at[pl.ds(base, BPW)])

    return run(user_ids, item_ids, uf, itf, ib)


def kernel(user_ids, item_ids, user_factors, item_factors, item_bias):
    return _dot_kernel(user_ids, item_ids, user_factors, item_factors,
                       item_bias)
